# baseline (speedup 1.0000x reference)
import jax
import jax.numpy as jnp
from jax.experimental import pallas as pl
from jax.experimental.pallas import tpu as pltpu


def _fused_kernel(x_ref, w1_ref, b1_ref, w2_ref, b2_ref, wg_ref, bg_ref,
                  wv_ref, out0_ref, out1_ref,
                  land_scr, w1b_scr, w2b_scr, wgb_scr, sem1, sem2):
    H = wgb_scr.shape[0]
    first = pl.program_id(0) == 0
    copy1 = pltpu.make_async_copy(w1_ref, land_scr, sem1)
    copy2 = pltpu.make_async_copy(w2_ref, land_scr, sem2)

    @pl.when(first)
    def _load_w1():
        copy1.start()
        copy1.wait()
        w1b_scr[...] = land_scr[...].astype(jnp.bfloat16)
        wgb_scr[...] = wg_ref[...].astype(jnp.bfloat16)
        copy2.start()

    x = x_ref[...]
    xb = x.astype(jnp.bfloat16)

    logits = jnp.dot(xb[:, :H], wgb_scr[...],
                     preferred_element_type=jnp.float32) + bg_ref[...]
    m = jnp.max(logits, axis=-1, keepdims=True)
    e = jnp.exp(logits - m)
    den = jnp.sum(e, axis=-1, keepdims=True)
    col = jax.lax.broadcasted_iota(jnp.int32, logits.shape, 1)
    am = jnp.argmax(logits, axis=-1)[:, None]
    m2 = jnp.max(jnp.where(col == am, -jnp.inf, logits),
                 axis=-1, keepdims=True)
    g = (1.0 + jnp.exp(m2 - m)) / den

    h = jnp.dot(xb, w1b_scr[...],
                preferred_element_type=jnp.float32).astype(jnp.bfloat16)
    h = jnp.maximum(h + b1_ref[...], jnp.bfloat16(0.0))

    @pl.when(first)
    def _load_w2():
        copy2.wait()
        w2b_scr[...] = land_scr[...].astype(jnp.bfloat16)

    eo = jnp.dot(h, w2b_scr[...],
                 preferred_element_type=jnp.float32) + b2_ref[...]

    wv = wv_ref[...]
    out0_ref[...] = eo[:, :H] * (g * wv[0:1, 0:1])
    out1_ref[...] = eo[:, H:] * wv[0:1, 1:2]


def kernel(vector, Wg, bg, Wf, bf, W1, b1, W2, b2, w):
    B, S, H2 = vector.shape
    H = H2 // 2
    E = Wg.shape[1]
    M = B * S
    BM = 1024
    x = vector.reshape(M, H2)

    grid = (M // BM,)
    b1b = b1.astype(jnp.bfloat16)

    out0, out1 = pl.pallas_call(
        _fused_kernel,
        grid=grid,
        in_specs=[
            pl.BlockSpec((BM, H2), lambda i: (i, 0)),
            pl.BlockSpec(memory_space=pl.ANY),
            pl.BlockSpec((1, H2), lambda i: (0, 0)),
            pl.BlockSpec(memory_space=pl.ANY),
            pl.BlockSpec((1, H2), lambda i: (0, 0)),
            pl.BlockSpec((H, E), lambda i: (0, 0)),
            pl.BlockSpec((1, E), lambda i: (0, 0)),
            pl.BlockSpec((1, 2), lambda i: (0, 0)),
        ],
        out_specs=[
            pl.BlockSpec((BM, H), lambda i: (i, 0)),
            pl.BlockSpec((BM, H), lambda i: (i, 0)),
        ],
        out_shape=[
            jax.ShapeDtypeStruct((M, H), jnp.float32),
            jax.ShapeDtypeStruct((M, H), jnp.float32),
        ],
        scratch_shapes=[
            pltpu.VMEM((H2, H2), jnp.float32),
            pltpu.VMEM((H2, H2), jnp.bfloat16),
            pltpu.VMEM((H2, H2), jnp.bfloat16),
            pltpu.VMEM((H, E), jnp.bfloat16),
            pltpu.SemaphoreType.DMA,
            pltpu.SemaphoreType.DMA,
        ],
        compiler_params=pltpu.CompilerParams(
            dimension_semantics=("arbitrary",),
        ),
    )(x, W1, b1b.reshape(1, H2), W2, b2.reshape(1, H2),
      Wg, bg.reshape(1, E), w.reshape(1, 2))

    return (out0.reshape(B, S, H), out1.reshape(B, S, H))


# speedup vs baseline: 1.0098x; 1.0098x over previous
import jax
import jax.numpy as jnp
from jax.experimental import pallas as pl
from jax.experimental.pallas import tpu as pltpu


def _fused_kernel(x_ref, w1_ref, b1_ref, w2_ref, b2_ref, wg_ref, bg_ref,
                  wv_ref, out0_ref, out1_ref, w1b_scr, w2b_scr, wgb_scr):
    H = wg_ref.shape[0]

    @pl.when(pl.program_id(0) == 0)
    def _cast_weights():
        w1b_scr[...] = w1_ref[...].astype(jnp.bfloat16)
        w2b_scr[...] = w2_ref[...].astype(jnp.bfloat16)
        wgb_scr[...] = wg_ref[...].astype(jnp.bfloat16)

    x = x_ref[...]
    xb = x.astype(jnp.bfloat16)

    logits = jnp.dot(xb[:, :H], wgb_scr[...],
                     preferred_element_type=jnp.float32) + bg_ref[...]
    m = jnp.max(logits, axis=-1, keepdims=True)
    e = jnp.exp(logits - m)
    den = jnp.sum(e, axis=-1, keepdims=True)
    col = jax.lax.broadcasted_iota(jnp.int32, logits.shape, 1)
    am = jnp.argmax(logits, axis=-1)[:, None]
    m2 = jnp.max(jnp.where(col == am, -jnp.inf, logits),
                 axis=-1, keepdims=True)
    g = (1.0 + jnp.exp(m2 - m)) / den

    h = jnp.dot(xb, w1b_scr[...],
                preferred_element_type=jnp.float32).astype(jnp.bfloat16)
    h = jnp.maximum(h + b1_ref[...], jnp.bfloat16(0.0))
    eo = jnp.dot(h, w2b_scr[...],
                 preferred_element_type=jnp.float32) + b2_ref[...]

    wv = wv_ref[...]
    out0_ref[...] = eo[:, :H] * (g * wv[0:1, 0:1])
    out1_ref[...] = eo[:, H:] * wv[0:1, 1:2]


def kernel(vector, Wg, bg, Wf, bf, W1, b1, W2, b2, w):
    B, S, H2 = vector.shape
    H = H2 // 2
    E = Wg.shape[1]
    M = B * S
    BM = 1024
    x = vector.reshape(M, H2)

    grid = (M // BM,)
    b1b = b1.astype(jnp.bfloat16)

    out0, out1 = pl.pallas_call(
        _fused_kernel,
        grid=grid,
        in_specs=[
            pl.BlockSpec((BM, H2), lambda i: (i, 0)),
            pl.BlockSpec((H2, H2), lambda i: (0, 0)),
            pl.BlockSpec((1, H2), lambda i: (0, 0)),
            pl.BlockSpec((H2, H2), lambda i: (0, 0)),
            pl.BlockSpec((1, H2), lambda i: (0, 0)),
            pl.BlockSpec((H, E), lambda i: (0, 0)),
            pl.BlockSpec((1, E), lambda i: (0, 0)),
            pl.BlockSpec((1, 2), lambda i: (0, 0)),
        ],
        out_specs=[
            pl.BlockSpec((BM, H), lambda i: (i, 0)),
            pl.BlockSpec((BM, H), lambda i: (i, 0)),
        ],
        out_shape=[
            jax.ShapeDtypeStruct((M, H), jnp.float32),
            jax.ShapeDtypeStruct((M, H), jnp.float32),
        ],
        scratch_shapes=[
            pltpu.VMEM((H2, H2), jnp.bfloat16),
            pltpu.VMEM((H2, H2), jnp.bfloat16),
            pltpu.VMEM((H, E), jnp.bfloat16),
        ],
        compiler_params=pltpu.CompilerParams(
            dimension_semantics=("arbitrary",),
        ),
    )(x, W1, b1b.reshape(1, H2), W2, b2.reshape(1, H2),
      Wg, bg.reshape(1, E), w.reshape(1, 2))

    return (out0.reshape(B, S, H), out1.reshape(B, S, H))
